# Initial kernel scaffold; baseline (speedup 1.0000x reference)
import jax
import jax.numpy as jnp
from jax.experimental import pallas as pl


def kernel(fm, counts):
    raise NotImplementedError("write your pallas kernel here")


# speedup vs baseline: 1.2063x; 1.2063x over previous
import functools

import jax
import jax.numpy as jnp
from jax import lax
from jax.experimental import pallas as pl
from jax.experimental.pallas import tpu as pltpu
from jax.experimental.pallas import tpu_sc as plsc

N, C, H, W = 8, 96, 64, 64
K2 = 9
P = H * W
PW = W + 2
TOTAL_ROWS = N * P * K2

NUM_WORKERS = 32
ROWS_PER_W = TOTAL_ROWS // NUM_WORKERS
CHUNK = 128
NCHUNKS = ROWS_PER_W // CHUNK
LANES = 16


def _sc_body(table_hbm, cnt_hbm, out_hbm, cnt_v, idx_v, rows_v, gsem):
    num_cores = 2
    wid = lax.axis_index("s") * num_cores + lax.axis_index("c")
    rbase = wid * ROWS_PER_W
    workers_per_batch = NUM_WORKERS // N
    nbase = (wid // workers_per_batch) * P
    cbase = (wid % workers_per_batch) * ROWS_PER_W

    pltpu.sync_copy(cnt_hbm.at[pl.ds(cbase, ROWS_PER_W)], cnt_v)

    def chunk_body(c, carry):
        vpw = jnp.full((LANES,), PW, jnp.int32)
        vone = jnp.full((LANES,), 1, jnp.int32)
        vzero = jnp.full((LANES,), 0, jnp.int32)
        vhmax = jnp.full((LANES,), H - 1, jnp.int32)
        vwmax = jnp.full((LANES,), W - 1, jnp.int32)
        vw = jnp.full((LANES,), W, jnp.int32)
        vnb = lax.broadcast_in_dim(nbase, (LANES,), ())
        for i in range(CHUNK // LANES):
            v = cnt_v[pl.ds(c * CHUNK + i * LANES, LANES)]
            ph = lax.div(v, vpw)
            pw = lax.sub(v, lax.mul(ph, vpw))
            hh = lax.max(lax.min(lax.sub(ph, vone), vhmax), vzero)
            ww = lax.max(lax.min(lax.sub(pw, vone), vwmax), vzero)
            idx_v[pl.ds(i * LANES, LANES)] = lax.add(
                lax.add(lax.mul(hh, vw), ww), vnb)
        pltpu.async_copy(table_hbm.at[idx_v], rows_v, gsem).wait()
        pltpu.sync_copy(rows_v, out_hbm.at[pl.ds(rbase + c * CHUNK, CHUNK)])
        return carry

    lax.fori_loop(0, NCHUNKS, chunk_body, 0)


@jax.jit
def _spatialflatten_sc(table, cnt):
    mesh = plsc.VectorSubcoreMesh(core_axis_name="c", subcore_axis_name="s")
    fn = pl.kernel(
        _sc_body,
        out_type=jax.ShapeDtypeStruct((TOTAL_ROWS, C), jnp.float32),
        mesh=mesh,
        scratch_types=[
            pltpu.VMEM((ROWS_PER_W,), jnp.int32),
            pltpu.VMEM((CHUNK,), jnp.int32),
            pltpu.VMEM((CHUNK, C), jnp.float32),
            pltpu.SemaphoreType.DMA,
        ],
        compiler_params=pltpu.CompilerParams(use_tc_tiling_on_sc=False),
    )
    return fn(table, cnt)


def kernel(fm, counts):
    table = jnp.transpose(fm, (0, 2, 3, 1)).reshape(N * P, C)
    cnt = counts.astype(jnp.int32).reshape(P * K2)
    out = _spatialflatten_sc(table, cnt)
    return out.reshape(N, P, K2 * C)


# speedup vs baseline: 1.4974x; 1.2413x over previous
import jax
import jax.numpy as jnp
from jax import lax
from jax.experimental import pallas as pl
from jax.experimental.pallas import tpu as pltpu
from jax.experimental.pallas import tpu_sc as plsc

N, C, H, W = 8, 96, 64, 64
K2 = 9
P = H * W
PW = W + 2
TOTAL_ROWS = N * P * K2

NUM_WORKERS = 32
ROWS_PER_W = TOTAL_ROWS // NUM_WORKERS
CHUNK = 128
NCHUNKS = ROWS_PER_W // CHUNK
NBUF = 4
LANES = 16


def _sc_body(table_hbm, cnt_hbm, out_hbm, cnt_v, idx_v, rows_v, *sems):
    gsems = sems[:NBUF]
    ssems = sems[NBUF:]
    num_cores = 2
    wid = lax.axis_index("s") * num_cores + lax.axis_index("c")
    rbase = wid * ROWS_PER_W
    workers_per_batch = NUM_WORKERS // N
    nbase = (wid // workers_per_batch) * P
    cbase = (wid % workers_per_batch) * ROWS_PER_W

    pltpu.sync_copy(cnt_hbm.at[pl.ds(cbase, ROWS_PER_W)], cnt_v)

    vpw = jnp.full((LANES,), PW, jnp.int32)
    vone = jnp.full((LANES,), 1, jnp.int32)
    vzero = jnp.full((LANES,), 0, jnp.int32)
    vhmax = jnp.full((LANES,), H - 1, jnp.int32)
    vwmax = jnp.full((LANES,), W - 1, jnp.int32)
    vw = jnp.full((LANES,), W, jnp.int32)
    vnb = lax.broadcast_in_dim(nbase, (LANES,), ())

    def decode(c, b):
        for i in range(CHUNK // LANES):
            v = cnt_v[pl.ds(c * CHUNK + i * LANES, LANES)]
            ph = lax.div(v, vpw)
            pw = lax.sub(v, lax.mul(ph, vpw))
            hh = lax.max(lax.min(lax.sub(ph, vone), vhmax), vzero)
            ww = lax.max(lax.min(lax.sub(pw, vone), vwmax), vzero)
            idx_v[b, pl.ds(i * LANES, LANES)] = lax.add(
                lax.add(lax.mul(hh, vw), ww), vnb)

    def gather_start(b):
        return pltpu.async_copy(
            table_hbm.at[idx_v.at[b]], rows_v.at[b], gsems[b])

    def gather_wait(b):
        pltpu.make_async_copy(
            table_hbm.at[idx_v.at[b]], rows_v.at[b], gsems[b]).wait()

    def scatter_start(b, g):
        return pltpu.async_copy(
            rows_v.at[b], out_hbm.at[pl.ds(rbase + g * CHUNK, CHUNK)],
            ssems[b])

    def scatter_wait(b):
        pltpu.make_async_copy(
            rows_v.at[b], out_hbm.at[pl.ds(rbase, CHUNK)], ssems[b]).wait()

    for b in range(NBUF):
        decode(b, b)
        gather_start(b)

    def outer_body(go, carry):
        for b in range(NBUF):
            g = go * NBUF + b
            gather_wait(b)
            scatter_start(b, g)
            decode(g + NBUF, b)
            scatter_wait(b)
            gather_start(b)
        return carry

    lax.fori_loop(0, (NCHUNKS - NBUF) // NBUF, outer_body, 0)

    for b in range(NBUF):
        g = NCHUNKS - NBUF + b
        gather_wait(b)
        scatter_start(b, g)
    for b in range(NBUF):
        scatter_wait(b)


@jax.jit
def _spatialflatten_sc(table, cnt):
    mesh = plsc.VectorSubcoreMesh(core_axis_name="c", subcore_axis_name="s")
    fn = pl.kernel(
        _sc_body,
        out_type=jax.ShapeDtypeStruct((TOTAL_ROWS, C), jnp.float32),
        mesh=mesh,
        scratch_types=[
            pltpu.VMEM((ROWS_PER_W,), jnp.int32),
            pltpu.VMEM((NBUF, CHUNK), jnp.int32),
            pltpu.VMEM((NBUF, CHUNK, C), jnp.float32),
        ] + [pltpu.SemaphoreType.DMA] * (2 * NBUF),
        compiler_params=pltpu.CompilerParams(use_tc_tiling_on_sc=False),
    )
    return fn(table, cnt)


def kernel(fm, counts):
    table = jnp.transpose(fm, (0, 2, 3, 1)).reshape(N * P, C)
    cnt = counts.astype(jnp.int32).reshape(P * K2)
    out = _spatialflatten_sc(table, cnt)
    return out.reshape(N, P, K2 * C)


# speedup vs baseline: 1.4976x; 1.0002x over previous
import jax
import jax.numpy as jnp
from jax import lax
from jax.experimental import pallas as pl
from jax.experimental.pallas import tpu as pltpu
from jax.experimental.pallas import tpu_sc as plsc

N, C, H, W = 8, 96, 64, 64
K2 = 9
P = H * W
PW = W + 2

NUM_WORKERS = 32
POS_PER_W = N * P // NUM_WORKERS
CPOS = 16
CROWS = CPOS * K2
GDMA = 72
NCHUNKS = POS_PER_W // CPOS
NBUF = 4
LANES = 16


def _sc_body(table_hbm, cnt_hbm, out_hbm, cnt_v, idx_v, rows_v, *sems):
    gsems = sems[:NBUF]
    ssems = sems[NBUF:]
    num_cores = 2
    wid = lax.axis_index("s") * num_cores + lax.axis_index("c")
    workers_per_batch = NUM_WORKERS // N
    n = wid // workers_per_batch
    nbase = n * P
    p0base = (wid % workers_per_batch) * POS_PER_W
    cntbase = p0base * K2

    pltpu.sync_copy(cnt_hbm.at[pl.ds(cntbase, POS_PER_W * K2)], cnt_v)

    vpw = jnp.full((LANES,), PW, jnp.int32)
    vone = jnp.full((LANES,), 1, jnp.int32)
    vzero = jnp.full((LANES,), 0, jnp.int32)
    vhmax = jnp.full((LANES,), H - 1, jnp.int32)
    vwmax = jnp.full((LANES,), W - 1, jnp.int32)
    vw = jnp.full((LANES,), W, jnp.int32)
    vnb = lax.broadcast_in_dim(nbase, (LANES,), ())

    def decode(c, b):
        for i in range(CROWS // LANES):
            v = cnt_v[pl.ds(c * CROWS + i * LANES, LANES)]
            ph = lax.div(v, vpw)
            pw = lax.sub(v, lax.mul(ph, vpw))
            hh = lax.max(lax.min(lax.sub(ph, vone), vhmax), vzero)
            ww = lax.max(lax.min(lax.sub(pw, vone), vwmax), vzero)
            idx_v[b, pl.ds(i * LANES, LANES)] = lax.add(
                lax.add(lax.mul(hh, vw), ww), vnb)

    out_rows = out_hbm
    rowbase = (nbase + p0base) * K2

    def gather_start(b):
        for j in range(CROWS // GDMA):
            pltpu.async_copy(
                table_hbm.at[idx_v.at[b, pl.ds(j * GDMA, GDMA)]],
                rows_v.at[b, pl.ds(j * GDMA, GDMA), :], gsems[b])

    def gather_wait(b):
        for j in range(CROWS // GDMA):
            pltpu.make_async_copy(
                table_hbm.at[idx_v.at[b, pl.ds(j * GDMA, GDMA)]],
                rows_v.at[b, pl.ds(j * GDMA, GDMA), :], gsems[b]).wait()

    def scatter_start(b, g):
        pltpu.async_copy(
            rows_v.at[b],
            out_rows.at[pl.ds(rowbase + g * CROWS, CROWS), :], ssems[b])

    def scatter_wait(b):
        pltpu.make_async_copy(
            rows_v.at[b],
            out_rows.at[pl.ds(rowbase, CROWS), :], ssems[b]).wait()

    for b in range(NBUF):
        decode(b, b)
        gather_start(b)

    def outer_body(go, carry):
        for b in range(NBUF):
            g = go * NBUF + b
            gather_wait(b)
            scatter_start(b, g)
            decode(g + NBUF, b)
            scatter_wait(b)
            gather_start(b)
        return carry

    lax.fori_loop(0, (NCHUNKS - NBUF) // NBUF, outer_body, 0)

    for b in range(NBUF):
        g = NCHUNKS - NBUF + b
        gather_wait(b)
        scatter_start(b, g)
    for b in range(NBUF):
        scatter_wait(b)


@jax.jit
def _spatialflatten_sc(table, cnt):
    mesh = plsc.VectorSubcoreMesh(core_axis_name="c", subcore_axis_name="s")
    fn = pl.kernel(
        _sc_body,
        out_type=jax.ShapeDtypeStruct((N * P * K2, C), jnp.float32),
        mesh=mesh,
        scratch_types=[
            pltpu.VMEM((POS_PER_W * K2,), jnp.int32),
            pltpu.VMEM((NBUF, CROWS), jnp.int32),
            pltpu.VMEM((NBUF, CROWS, C), jnp.float32),
        ] + [pltpu.SemaphoreType.DMA] * (2 * NBUF),
        compiler_params=pltpu.CompilerParams(use_tc_tiling_on_sc=False),
    )
    return fn(table, cnt).reshape(N, P, K2 * C)


def kernel(fm, counts):
    table = jnp.transpose(fm, (0, 2, 3, 1)).reshape(N * P, C)
    cnt = counts.astype(jnp.int32).reshape(P * K2)
    return _spatialflatten_sc(table, cnt)
